# baseline (speedup 1.0000x reference)
import functools

import jax
import jax.numpy as jnp
from jax import lax
from jax.experimental import pallas as pl
from jax.experimental.pallas import tpu as pltpu
from jax.experimental.pallas import tpu_sc as plsc

_B, _T, _C = 4, 8192, 1024
_NW = 32
_ROWS_PER_W = _B * _T // _NW
_CHUNK = 16
_CW = _CHUNK * _C
_NCHUNK = _ROWS_PER_W // _CHUNK

_mesh = plsc.VectorSubcoreMesh(core_axis_name="c", subcore_axis_name="s")


@functools.partial(
    pl.kernel,
    mesh=_mesh,
    out_type=jax.ShapeDtypeStruct((_B * _T * _C,), jnp.float32),
    scratch_types=[
        pltpu.VMEM((_CW,), jnp.float32),
        pltpu.VMEM((_CW,), jnp.float32),
    ],
)
def _sc_add(x_hbm, pe_hbm, out_hbm, xbuf, pebuf):
    wid = lax.axis_index("s") * 2 + lax.axis_index("c")
    x0 = wid * (_ROWS_PER_W * _C)
    pe0 = (wid % (_T // _ROWS_PER_W)) * (_ROWS_PER_W * _C)

    def chunk_body(j, carry):
        xoff = x0 + j * _CW
        poff = pe0 + j * _CW
        pltpu.sync_copy(x_hbm.at[pl.ds(xoff, _CW)], xbuf)
        pltpu.sync_copy(pe_hbm.at[pl.ds(poff, _CW)], pebuf)

        @plsc.parallel_loop(0, _CW, 16, unroll=8)
        def _(i):
            s = pl.ds(i, 16)
            xbuf[s] = xbuf[s] + pebuf[s]

        pltpu.sync_copy(xbuf, out_hbm.at[pl.ds(xoff, _CW)])
        return carry

    lax.fori_loop(0, _NCHUNK, chunk_body, 0)


def kernel(x, pos_emb):
    B, T, C = x.shape
    out = _sc_add(x.reshape(-1), pos_emb.reshape(-1))
    return out.reshape(B, T, C)


# speedup vs baseline: 2.6553x; 2.6553x over previous
import functools

import jax
import jax.numpy as jnp
from jax import lax
from jax.experimental import pallas as pl
from jax.experimental.pallas import tpu as pltpu
from jax.experimental.pallas import tpu_sc as plsc

_B, _T, _C = 4, 8192, 1024
_TCB = _B - 1
_TB = 512

_NW = 32
_ROWS_PER_W = _T // _NW
_CHUNK = 16
_NCHUNK = _ROWS_PER_W // _CHUNK

_mesh = plsc.VectorSubcoreMesh(core_axis_name="c", subcore_axis_name="s")


def _tc_body(x_ref, pe_ref, o_ref):
    o_ref[...] = x_ref[...] + pe_ref[...][None, :, :]


@functools.partial(
    pl.kernel,
    mesh=_mesh,
    out_type=jax.ShapeDtypeStruct((_T, _C), jnp.float32),
    scratch_types=[
        pltpu.VMEM((_CHUNK, _C), jnp.float32),
        pltpu.VMEM((_CHUNK, _C), jnp.float32),
    ],
    compiler_params=pltpu.CompilerParams(use_tc_tiling_on_sc=True),
)
def _sc_add_b3(x_hbm, pe_hbm, out_hbm, xbuf, pebuf):
    wid = lax.axis_index("s") * 2 + lax.axis_index("c")
    row0 = wid * _ROWS_PER_W

    def chunk_body(j, carry):
        r = row0 + j * _CHUNK
        pltpu.sync_copy(x_hbm.at[_B - 1, pl.ds(r, _CHUNK)], xbuf)
        pltpu.sync_copy(pe_hbm.at[pl.ds(r, _CHUNK)], pebuf)

        @plsc.parallel_loop(0, _CHUNK * _C, 16, unroll=8)
        def _(i):
            row = i // _C
            col = i % _C
            xbuf[row, pl.ds(col, 16)] = (
                xbuf[row, pl.ds(col, 16)] + pebuf[row, pl.ds(col, 16)]
            )

        pltpu.sync_copy(xbuf, out_hbm.at[pl.ds(r, _CHUNK)])
        return carry

    lax.fori_loop(0, _NCHUNK, chunk_body, 0)


def kernel(x, pos_emb):
    B, T, C = x.shape
    out3 = _sc_add_b3(x, pos_emb)
    out012 = pl.pallas_call(
        _tc_body,
        grid=(T // _TB,),
        in_specs=[
            pl.BlockSpec((_TCB, _TB, C), lambda i: (0, i, 0)),
            pl.BlockSpec((_TB, C), lambda i: (i, 0)),
        ],
        out_specs=pl.BlockSpec((_TCB, _TB, C), lambda i: (0, i, 0)),
        out_shape=jax.ShapeDtypeStruct((_TCB, T, C), x.dtype),
        compiler_params=pltpu.CompilerParams(
            dimension_semantics=("arbitrary",),
        ),
    )(x, pos_emb)
    return jnp.concatenate([out012, out3[None]], axis=0)


# speedup vs baseline: 6.1066x; 2.2998x over previous
import jax
import jax.numpy as jnp
from jax.experimental import pallas as pl
from jax.experimental.pallas import tpu as pltpu

_TB = 256


def _add_body(x_ref, pe_ref, o_ref):
    o_ref[...] = x_ref[...] + pe_ref[...][None, :, :]


def kernel(x, pos_emb):
    B, T, C = x.shape
    grid = (T // _TB,)
    return pl.pallas_call(
        _add_body,
        grid=grid,
        in_specs=[
            pl.BlockSpec((B, _TB, C), lambda i: (0, i, 0)),
            pl.BlockSpec((_TB, C), lambda i: (i, 0)),
        ],
        out_specs=pl.BlockSpec((B, _TB, C), lambda i: (0, i, 0)),
        out_shape=jax.ShapeDtypeStruct((B, T, C), x.dtype),
        compiler_params=pltpu.CompilerParams(
            dimension_semantics=("arbitrary",),
        ),
    )(x, pos_emb)
